# baseline (speedup 1.0000x reference)
import functools

import jax
import jax.numpy as jnp
from jax.experimental import pallas as pl
from jax.experimental.pallas import tpu as pltpu

K1, S1 = 10, 5
K2, S2 = 10, 7
LANE = 128
HB = 64


def _round_up(x, m):
    return (x + m - 1) // m * m


def _make_body(T1, E, C2L, OUTW, NCLS):
    def body(x_ref, w1_ref, w2_ref, wfc_ref, bias_ref, out_ref, h_ref):
        w1b = w1_ref[...]
        for t in range(T1):
            xs = x_ref[t * S1 * E:t * S1 * E + K1 * E, :].astype(jnp.bfloat16)
            d = jnp.dot(w1b, xs, preferred_element_type=jnp.float32)
            b1c = bias_ref[t * HB:(t + 1) * HB, 0:1]
            h_ref[t * HB:(t + 1) * HB, :] = jnp.maximum(d + b1c, 0.0
                                                        ).astype(jnp.bfloat16)
        b2c = bias_ref[0:C2L, 1:2]
        bfcc = bias_ref[0:OUTW, 2:3]
        o2 = jnp.dot(w2_ref[...], h_ref[...], preferred_element_type=jnp.float32)
        o2 = jnp.maximum(o2 + b2c, 0.0).astype(jnp.bfloat16)
        r = (jnp.dot(wfc_ref[...], o2,
                     preferred_element_type=jnp.float32) + bfcc)
        out_ref[...] = jnp.transpose(r)[:, :NCLS]
    return body


@functools.partial(jax.jit, static_argnames=("batch_block",))
def _impl(x_emb, w1, b1, w2, b2, wfc, bfc, *, batch_block=2048):
    B, L, E = x_emb.shape
    H = w1.shape[0]
    C2 = w2.shape[0]
    n_cls = wfc.shape[0]
    L1 = (L - K1) // S1 + 1
    L2 = (L1 - K2) // S2 + 1
    T1 = (L2 - 1) * S2 + K2
    XW = L * E
    HT = T1 * HB
    C2P = _round_up(C2, LANE)
    C2L = L2 * C2P
    OUTW = 8
    cdtype = jnp.bfloat16
    assert S1 * E % 8 == 0 and H <= HB

    w1f = jnp.transpose(w1, (0, 2, 1)).reshape(H, K1 * E)
    w1f = jnp.pad(w1f, ((0, HB - H), (0, 0))).astype(cdtype)

    w2c = jnp.transpose(w2, (0, 2, 1))
    w2c = jnp.pad(w2c, ((0, C2P - C2), (0, 0), (0, HB - H)))
    w2c = w2c.reshape(C2P, K2 * HB)
    W2T = jnp.concatenate(
        [jnp.pad(w2c, ((0, 0), (t2 * S2 * HB, (T1 - t2 * S2 - K2) * HB)))
         for t2 in range(L2)], axis=0).astype(cdtype)

    wfc_r = jnp.transpose(wfc.reshape(n_cls, C2, L2), (0, 2, 1))
    WfcT = jnp.pad(wfc_r, ((0, 0), (0, 0), (0, C2P - C2))).reshape(n_cls, C2L)
    WfcT = jnp.pad(WfcT, ((0, OUTW - n_cls), (0, 0))).astype(cdtype)

    c0 = jnp.tile(jnp.pad(b1.astype(jnp.float32), (0, HB - H)), T1)
    c1 = jnp.pad(jnp.tile(jnp.pad(b2.astype(jnp.float32), (0, C2P - C2)), L2),
                 (0, HT - C2L))
    c2 = jnp.pad(bfc.astype(jnp.float32), (0, HT - n_cls))
    biasmat = jnp.pad(jnp.stack([c0, c1, c2], axis=1), ((0, 0), (0, 5)))

    xt = jnp.transpose(x_emb, (1, 2, 0)).reshape(XW, B)
    BN = min(batch_block, _round_up(B, LANE))
    nb = pl.cdiv(B, BN)
    Bp = nb * BN
    if Bp != B:
        xt = jnp.pad(xt, ((0, 0), (0, Bp - B)))

    out = pl.pallas_call(
        _make_body(T1, E, C2L, OUTW, n_cls),
        out_shape=jax.ShapeDtypeStruct((Bp, n_cls), jnp.float32),
        grid=(nb,),
        in_specs=[
            pl.BlockSpec((XW, BN), lambda i: (0, i)),
            pl.BlockSpec((HB, K1 * E), lambda i: (0, 0)),
            pl.BlockSpec((C2L, HT), lambda i: (0, 0)),
            pl.BlockSpec((OUTW, C2L), lambda i: (0, 0)),
            pl.BlockSpec((HT, 8), lambda i: (0, 0)),
        ],
        out_specs=pl.BlockSpec((BN, n_cls), lambda i: (i, 0)),
        scratch_shapes=[pltpu.VMEM((HT, BN), cdtype)],
        compiler_params=pltpu.CompilerParams(
            dimension_semantics=("parallel",),
            vmem_limit_bytes=64 * 1024 * 1024,
        ),
    )(xt, w1f, W2T, WfcT, biasmat)

    return out[:B]


def kernel(x_emb, w1, b1, w2, b2, wfc, bfc):
    return _impl(x_emb, w1, b1, w2, b2, wfc, bfc)


# speedup vs baseline: 1.1297x; 1.1297x over previous
import functools

import jax
import jax.numpy as jnp
from jax.experimental import pallas as pl
from jax.experimental.pallas import tpu as pltpu

K1, S1 = 10, 5
K2, S2 = 10, 7
LANE = 128
HB = 64


def _round_up(x, m):
    return (x + m - 1) // m * m


def _make_body(T1, E, C2L, OUTW, BNH):
    def body(x0_ref, x1_ref, w1_ref, w2_ref, wfc_ref, bias_ref, out_ref,
             h0_ref, h1_ref):
        w1b = w1_ref[...]
        b2c = bias_ref[0:C2L, 1:2]
        bfcc = bias_ref[0:OUTW, 2:3]
        for j, (xr, hr) in enumerate(((x0_ref, h0_ref), (x1_ref, h1_ref))):
            for t in range(T1):
                xs = xr[t * S1 * E:t * S1 * E + K1 * E, :].astype(jnp.bfloat16)
                d = jnp.dot(w1b, xs, preferred_element_type=jnp.float32)
                b1c = bias_ref[t * HB:(t + 1) * HB, 0:1]
                hr[t * HB:(t + 1) * HB, :] = jnp.maximum(d + b1c, 0.0
                                                         ).astype(jnp.bfloat16)
            o2 = jnp.dot(w2_ref[...], hr[...], preferred_element_type=jnp.float32)
            o2 = jnp.maximum(o2 + b2c, 0.0).astype(jnp.bfloat16)
            out_ref[:, j * BNH:(j + 1) * BNH] = (
                jnp.dot(wfc_ref[...], o2, preferred_element_type=jnp.float32)
                + bfcc)
    return body


@functools.partial(jax.jit, static_argnames=("batch_block",))
def _impl(x_emb, w1, b1, w2, b2, wfc, bfc, *, batch_block=2048):
    B, L, E = x_emb.shape
    H = w1.shape[0]
    C2 = w2.shape[0]
    n_cls = wfc.shape[0]
    L1 = (L - K1) // S1 + 1
    L2 = (L1 - K2) // S2 + 1
    T1 = (L2 - 1) * S2 + K2
    XW = L * E
    HT = T1 * HB
    C2P = _round_up(C2, LANE)
    C2L = L2 * C2P
    OUTW = 8
    cdtype = jnp.bfloat16
    assert S1 * E % 8 == 0 and H <= HB

    w1f = jnp.transpose(w1, (0, 2, 1)).reshape(H, K1 * E)
    w1f = jnp.pad(w1f, ((0, HB - H), (0, 0))).astype(cdtype)

    w2c = jnp.transpose(w2, (0, 2, 1))
    w2c = jnp.pad(w2c, ((0, C2P - C2), (0, 0), (0, HB - H)))
    w2c = w2c.reshape(C2P, K2 * HB)
    W2T = jnp.concatenate(
        [jnp.pad(w2c, ((0, 0), (t2 * S2 * HB, (T1 - t2 * S2 - K2) * HB)))
         for t2 in range(L2)], axis=0).astype(cdtype)

    wfc_r = jnp.transpose(wfc.reshape(n_cls, C2, L2), (0, 2, 1))
    WfcT = jnp.pad(wfc_r, ((0, 0), (0, 0), (0, C2P - C2))).reshape(n_cls, C2L)
    WfcT = jnp.pad(WfcT, ((0, OUTW - n_cls), (0, 0))).astype(cdtype)

    c0 = jnp.tile(jnp.pad(b1.astype(jnp.float32), (0, HB - H)), T1)
    c1 = jnp.pad(jnp.tile(jnp.pad(b2.astype(jnp.float32), (0, C2P - C2)), L2),
                 (0, HT - C2L))
    c2 = jnp.pad(bfc.astype(jnp.float32), (0, HT - n_cls))
    biasmat = jnp.pad(jnp.stack([c0, c1, c2], axis=1), ((0, 0), (0, 5)))

    xt = jnp.transpose(x_emb, (1, 2, 0)).reshape(XW, B)
    BN = min(batch_block, _round_up(B, LANE))
    nb = pl.cdiv(B, BN)
    Bp = nb * BN
    if Bp != B:
        xt = jnp.pad(xt, ((0, 0), (0, Bp - B)))

    out = pl.pallas_call(
        _make_body(T1, E, C2L, OUTW, BN // 2),
        out_shape=jax.ShapeDtypeStruct((OUTW, Bp), jnp.float32),
        grid=(nb,),
        in_specs=[
            pl.BlockSpec((XW, BN // 2), lambda i: (0, 2 * i)),
            pl.BlockSpec((XW, BN // 2), lambda i: (0, 2 * i + 1)),
            pl.BlockSpec((HB, K1 * E), lambda i: (0, 0)),
            pl.BlockSpec((C2L, HT), lambda i: (0, 0)),
            pl.BlockSpec((OUTW, C2L), lambda i: (0, 0)),
            pl.BlockSpec((HT, 8), lambda i: (0, 0)),
        ],
        out_specs=pl.BlockSpec((OUTW, BN), lambda i: (0, i)),
        scratch_shapes=[pltpu.VMEM((HT, BN // 2), cdtype),
                        pltpu.VMEM((HT, BN // 2), cdtype)],
        compiler_params=pltpu.CompilerParams(
            dimension_semantics=("parallel",),
            vmem_limit_bytes=64 * 1024 * 1024,
        ),
    )(xt, xt, w1f, W2T, WfcT, biasmat)

    return jnp.transpose(out[:n_cls, :B])


def kernel(x_emb, w1, b1, w2, b2, wfc, bfc):
    return _impl(x_emb, w1, b1, w2, b2, wfc, bfc)


# speedup vs baseline: 1.1824x; 1.0466x over previous
import functools

import jax
import jax.numpy as jnp
from jax.experimental import pallas as pl
from jax.experimental.pallas import tpu as pltpu

K1, S1 = 10, 5
K2, S2 = 10, 7
LANE = 128
HB = 64


def _round_up(x, m):
    return (x + m - 1) // m * m


def _make_body(T1, E, C2L, OUTW):
    def body(x_ref, w1_ref, w2_ref, wfc_ref, bias_ref, out_ref, h_ref):
        w1b = w1_ref[...]
        for t in range(T1):
            xs = x_ref[t * S1 * E:t * S1 * E + K1 * E, :].astype(jnp.bfloat16)
            d = jnp.dot(w1b, xs, preferred_element_type=jnp.float32)
            b1c = bias_ref[t * HB:(t + 1) * HB, 0:1]
            h_ref[t * HB:(t + 1) * HB, :] = jnp.maximum(d + b1c, 0.0
                                                        ).astype(jnp.bfloat16)
        b2c = bias_ref[0:C2L, 1:2]
        bfcc = bias_ref[0:OUTW, 2:3]
        o2 = jnp.dot(w2_ref[...], h_ref[...], preferred_element_type=jnp.float32)
        o2 = jnp.maximum(o2 + b2c, 0.0).astype(jnp.bfloat16)
        out_ref[...] = (jnp.dot(wfc_ref[...], o2,
                                preferred_element_type=jnp.float32) + bfcc)
    return body


@functools.partial(jax.jit, static_argnames=("batch_block",))
def _impl(x_emb, w1, b1, w2, b2, wfc, bfc, *, batch_block=2048):
    B, L, E = x_emb.shape
    H = w1.shape[0]
    C2 = w2.shape[0]
    n_cls = wfc.shape[0]
    L1 = (L - K1) // S1 + 1
    L2 = (L1 - K2) // S2 + 1
    T1 = (L2 - 1) * S2 + K2
    XW = L * E
    HT = T1 * HB
    C2P = _round_up(C2, LANE)
    C2L = L2 * C2P
    OUTW = 8
    cdtype = jnp.bfloat16
    assert S1 * E % 8 == 0 and H <= HB

    w1f = jnp.transpose(w1, (0, 2, 1)).reshape(H, K1 * E)
    w1f = jnp.pad(w1f, ((0, HB - H), (0, 0))).astype(cdtype)

    w2c = jnp.transpose(w2, (0, 2, 1))
    w2c = jnp.pad(w2c, ((0, C2P - C2), (0, 0), (0, HB - H)))
    w2c = w2c.reshape(C2P, K2 * HB)
    W2T = jnp.concatenate(
        [jnp.pad(w2c, ((0, 0), (t2 * S2 * HB, (T1 - t2 * S2 - K2) * HB)))
         for t2 in range(L2)], axis=0).astype(cdtype)

    wfc_r = jnp.transpose(wfc.reshape(n_cls, C2, L2), (0, 2, 1))
    WfcT = jnp.pad(wfc_r, ((0, 0), (0, 0), (0, C2P - C2))).reshape(n_cls, C2L)
    WfcT = jnp.pad(WfcT, ((0, OUTW - n_cls), (0, 0))).astype(cdtype)

    c0 = jnp.tile(jnp.pad(b1.astype(jnp.float32), (0, HB - H)), T1)
    c1 = jnp.pad(jnp.tile(jnp.pad(b2.astype(jnp.float32), (0, C2P - C2)), L2),
                 (0, HT - C2L))
    c2 = jnp.pad(bfc.astype(jnp.float32), (0, HT - n_cls))
    biasmat = jnp.pad(jnp.stack([c0, c1, c2], axis=1), ((0, 0), (0, 5)))

    XR = (T1 - 1) * S1 * E + K1 * E
    xt = jnp.transpose(x_emb, (1, 2, 0)).reshape(XW, B)
    BN = min(batch_block, _round_up(B, LANE))
    nb = pl.cdiv(B, BN)
    Bp = nb * BN
    if Bp != B:
        xt = jnp.pad(xt, ((0, 0), (0, Bp - B)))

    out = pl.pallas_call(
        _make_body(T1, E, C2L, OUTW),
        out_shape=jax.ShapeDtypeStruct((OUTW, Bp), jnp.float32),
        grid=(nb,),
        in_specs=[
            pl.BlockSpec((XR, BN), lambda i: (0, i)),
            pl.BlockSpec((HB, K1 * E), lambda i: (0, 0)),
            pl.BlockSpec((C2L, HT), lambda i: (0, 0)),
            pl.BlockSpec((OUTW, C2L), lambda i: (0, 0)),
            pl.BlockSpec((HT, 8), lambda i: (0, 0)),
        ],
        out_specs=pl.BlockSpec((OUTW, BN), lambda i: (0, i)),
        scratch_shapes=[pltpu.VMEM((HT, BN), cdtype)],
        compiler_params=pltpu.CompilerParams(
            dimension_semantics=("parallel",),
            vmem_limit_bytes=64 * 1024 * 1024,
        ),
    )(xt, w1f, W2T, WfcT, biasmat)

    return jnp.transpose(out[:n_cls, :B])


def kernel(x_emb, w1, b1, w2, b2, wfc, bfc):
    return _impl(x_emb, w1, b1, w2, b2, wfc, bfc)


# speedup vs baseline: 1.2412x; 1.0497x over previous
import functools

import jax
import jax.numpy as jnp
from jax.experimental import pallas as pl
from jax.experimental.pallas import tpu as pltpu

K1, S1 = 10, 5
K2, S2 = 10, 7
LANE = 128
HB = 64


def _round_up(x, m):
    return (x + m - 1) // m * m


def _make_body(T1, E, C2L, OUTW):
    def body(x_ref, w_ref, bias_ref, out_ref, h_ref):
        w2w = w_ref[0:C2L, :]
        wfcw = w_ref[C2L:C2L + OUTW, 0:C2L]
        w1b = w_ref[C2L + OUTW:C2L + OUTW + HB, 0:K1 * E]
        for t in range(T1):
            xs = x_ref[t * S1 * E:t * S1 * E + K1 * E, :].astype(jnp.bfloat16)
            d = jnp.dot(w1b, xs, preferred_element_type=jnp.float32)
            b1c = bias_ref[t * HB:(t + 1) * HB, 0:1]
            h_ref[t * HB:(t + 1) * HB, :] = jnp.maximum(d + b1c, 0.0
                                                        ).astype(jnp.bfloat16)
        b2c = bias_ref[0:C2L, 1:2]
        bfcc = bias_ref[0:OUTW, 2:3]
        o2 = jnp.dot(w2w, h_ref[...], preferred_element_type=jnp.float32)
        o2 = jnp.maximum(o2 + b2c, 0.0).astype(jnp.bfloat16)
        out_ref[...] = (jnp.dot(wfcw, o2,
                                preferred_element_type=jnp.float32) + bfcc)
    return body


@functools.partial(jax.jit, static_argnames=("batch_block",))
def _impl(x_emb, w1, b1, w2, b2, wfc, bfc, *, batch_block=2048):
    B, L, E = x_emb.shape
    H = w1.shape[0]
    C2 = w2.shape[0]
    n_cls = wfc.shape[0]
    L1 = (L - K1) // S1 + 1
    L2 = (L1 - K2) // S2 + 1
    T1 = (L2 - 1) * S2 + K2
    XW = L * E
    HT = T1 * HB
    C2P = _round_up(C2, LANE)
    C2L = L2 * C2P
    OUTW = 8
    cdtype = jnp.bfloat16
    assert S1 * E % 8 == 0 and H <= HB

    w1f = jnp.transpose(w1, (0, 2, 1)).reshape(H, K1 * E)
    w1f = jnp.pad(w1f, ((0, HB - H), (0, 0))).astype(cdtype)

    w2c = jnp.transpose(w2, (0, 2, 1))
    w2c = jnp.pad(w2c, ((0, C2P - C2), (0, 0), (0, HB - H)))
    w2c = w2c.reshape(C2P, K2 * HB)
    W2T = jnp.concatenate(
        [jnp.pad(w2c, ((0, 0), (t2 * S2 * HB, (T1 - t2 * S2 - K2) * HB)))
         for t2 in range(L2)], axis=0).astype(cdtype)

    wfc_r = jnp.transpose(wfc.reshape(n_cls, C2, L2), (0, 2, 1))
    WfcT = jnp.pad(wfc_r, ((0, 0), (0, 0), (0, C2P - C2))).reshape(n_cls, C2L)
    WfcT = jnp.pad(WfcT, ((0, OUTW - n_cls), (0, 0))).astype(cdtype)

    Wpack = jnp.concatenate([
        W2T,
        jnp.pad(WfcT, ((0, 0), (0, HT - C2L))),
        jnp.pad(w1f, ((0, 0), (0, HT - K1 * E))),
    ], axis=0)

    c0 = jnp.tile(jnp.pad(b1.astype(jnp.float32), (0, HB - H)), T1)
    c1 = jnp.pad(jnp.tile(jnp.pad(b2.astype(jnp.float32), (0, C2P - C2)), L2),
                 (0, HT - C2L))
    c2 = jnp.pad(bfc.astype(jnp.float32), (0, HT - n_cls))
    biasmat = jnp.pad(jnp.stack([c0, c1, c2], axis=1), ((0, 0), (0, 5)))

    XR = (T1 - 1) * S1 * E + K1 * E
    xt = jnp.transpose(x_emb, (1, 2, 0)).reshape(XW, B)
    BN = min(batch_block, _round_up(B, LANE))
    nb = pl.cdiv(B, BN)
    Bp = nb * BN
    if Bp != B:
        xt = jnp.pad(xt, ((0, 0), (0, Bp - B)))

    out = pl.pallas_call(
        _make_body(T1, E, C2L, OUTW),
        out_shape=jax.ShapeDtypeStruct((OUTW, Bp), jnp.float32),
        grid=(nb,),
        in_specs=[
            pl.BlockSpec((XR, BN), lambda i: (0, i)),
            pl.BlockSpec((C2L + OUTW + HB, HT), lambda i: (0, 0)),
            pl.BlockSpec((HT, 8), lambda i: (0, 0)),
        ],
        out_specs=pl.BlockSpec((OUTW, BN), lambda i: (0, i)),
        scratch_shapes=[pltpu.VMEM((HT, BN), cdtype)],
        compiler_params=pltpu.CompilerParams(
            dimension_semantics=("parallel",),
            vmem_limit_bytes=64 * 1024 * 1024,
        ),
    )(xt, Wpack, biasmat)

    return jnp.transpose(out[:n_cls, :B])


def kernel(x_emb, w1, b1, w2, b2, wfc, bfc):
    return _impl(x_emb, w1, b1, w2, b2, wfc, bfc)


# speedup vs baseline: 1.4169x; 1.1416x over previous
import functools

import jax
import jax.numpy as jnp
from jax.experimental import pallas as pl
from jax.experimental.pallas import tpu as pltpu

K1, S1 = 10, 5
K2, S2 = 10, 7
LANE = 128
HB = 64


def _round_up(x, m):
    return (x + m - 1) // m * m


def _make_body(T1, E, C2L, OUTW, H, C2, NCLS):
    def body(x_ref, w_ref, b1_ref, b2_ref, bfc_ref, out_ref, h_ref):
        w2w = w_ref[0:C2L, :]
        wfcw = w_ref[C2L:C2L + OUTW, 0:C2L]
        w1v = w_ref[C2L + OUTW:C2L + OUTW + K1 * E, 0:HB]
        b1c = jnp.concatenate(
            [jnp.transpose(b1_ref[...]), jnp.zeros((HB - H, 1), jnp.float32)],
            axis=0)
        b2c = jnp.concatenate(
            [jnp.transpose(b2_ref[...]), jnp.zeros((C2L - C2, 1), jnp.float32)],
            axis=0)
        bfcc = jnp.concatenate(
            [jnp.transpose(bfc_ref[...]),
             jnp.zeros((OUTW - NCLS, 1), jnp.float32)], axis=0)
        for t in range(T1):
            xs = x_ref[t * S1 * E:t * S1 * E + K1 * E, :].astype(jnp.bfloat16)
            d = jax.lax.dot_general(w1v, xs, (((0,), (0,)), ((), ())),
                                    preferred_element_type=jnp.float32)
            h_ref[t * HB:(t + 1) * HB, :] = jnp.maximum(d + b1c, 0.0
                                                        ).astype(jnp.bfloat16)
        o2 = jnp.dot(w2w, h_ref[...], preferred_element_type=jnp.float32)
        o2 = jnp.maximum(o2 + b2c, 0.0).astype(jnp.bfloat16)
        out_ref[...] = (jnp.dot(wfcw, o2,
                                preferred_element_type=jnp.float32) + bfcc)
    return body


@functools.partial(jax.jit, static_argnames=("batch_block",))
def _impl(x_emb, w1, b1, w2, b2, wfc, bfc, *, batch_block=2048):
    B, L, E = x_emb.shape
    H = w1.shape[0]
    C2 = w2.shape[0]
    n_cls = wfc.shape[0]
    L1 = (L - K1) // S1 + 1
    L2 = (L1 - K2) // S2 + 1
    T1 = (L2 - 1) * S2 + K2
    XW = L * E
    HT = T1 * HB
    C2P = _round_up(C2, LANE)
    C2L = L2 * C2P
    OUTW = 8
    cdtype = jnp.bfloat16
    assert S1 * E % 8 == 0 and H <= HB

    w1n = jnp.transpose(w1, (2, 1, 0)).reshape(K1 * E, H)
    w1n = jnp.pad(w1n, ((0, 0), (0, HB - H))).astype(cdtype)

    w2c = jnp.transpose(w2, (0, 2, 1))
    w2c = jnp.pad(w2c, ((0, C2P - C2), (0, 0), (0, HB - H)))
    w2c = w2c.reshape(C2P, K2 * HB)
    W2T = jnp.concatenate(
        [jnp.pad(w2c, ((0, 0), (t2 * S2 * HB, (T1 - t2 * S2 - K2) * HB)))
         for t2 in range(L2)], axis=0).astype(cdtype)

    wfc_r = jnp.transpose(wfc.reshape(n_cls, C2, L2), (0, 2, 1))
    WfcT = jnp.pad(wfc_r, ((0, 0), (0, 0), (0, C2P - C2))).reshape(n_cls, C2L)
    WfcT = jnp.pad(WfcT, ((0, OUTW - n_cls), (0, 0))).astype(cdtype)

    Wpack = jnp.concatenate([
        W2T,
        jnp.pad(WfcT, ((0, 0), (0, HT - C2L))),
        jnp.pad(w1n, ((0, 0), (0, HT - HB))),
    ], axis=0)

    XR = (T1 - 1) * S1 * E + K1 * E
    xt = jnp.transpose(x_emb, (1, 2, 0)).reshape(XW, B)
    BN = min(batch_block, _round_up(B, LANE))
    nb = pl.cdiv(B, BN)
    Bp = nb * BN
    if Bp != B:
        xt = jnp.pad(xt, ((0, 0), (0, Bp - B)))

    out = pl.pallas_call(
        _make_body(T1, E, C2L, OUTW, H, C2, n_cls),
        out_shape=jax.ShapeDtypeStruct((OUTW, Bp), jnp.float32),
        grid=(nb,),
        in_specs=[
            pl.BlockSpec((XR, BN), lambda i: (0, i)),
            pl.BlockSpec((C2L + OUTW + K1 * E, HT), lambda i: (0, 0)),
            pl.BlockSpec((1, H), lambda i: (0, 0)),
            pl.BlockSpec((1, C2), lambda i: (0, 0)),
            pl.BlockSpec((1, n_cls), lambda i: (0, 0)),
        ],
        out_specs=pl.BlockSpec((OUTW, BN), lambda i: (0, i)),
        scratch_shapes=[pltpu.VMEM((HT, BN), cdtype)],
        compiler_params=pltpu.CompilerParams(
            dimension_semantics=("parallel",),
            vmem_limit_bytes=64 * 1024 * 1024,
        ),
    )(xt, Wpack, b1.reshape(1, H).astype(jnp.float32),
      b2.reshape(1, C2).astype(jnp.float32),
      bfc.reshape(1, n_cls).astype(jnp.float32))

    return jnp.transpose(out[:n_cls, :B])


def kernel(x_emb, w1, b1, w2, b2, wfc, bfc):
    return _impl(x_emb, w1, b1, w2, b2, wfc, bfc)
